# baseline (speedup 1.0000x reference)
import functools

import jax
import jax.numpy as jnp
from jax.experimental import pallas as pl
from jax.experimental.pallas import tpu as pltpu


def _make_body(nblk, blk):
    def _body(seq_ref, w_ref, b_ref, a_ref, adj_ref, out_ref,
              fts_ref, abuf_ref, obuf_ref, asem, osem):
        pltpu.make_async_copy(adj_ref.at[pl.ds(0, blk), :],
                              abuf_ref.at[0], asem.at[0]).start()
        if nblk > 1:
            pltpu.make_async_copy(adj_ref.at[pl.ds(blk, blk), :],
                                  abuf_ref.at[1], asem.at[1]).start()

        fts_ref[...] = jax.lax.dot_general(
            seq_ref[...], w_ref[...],
            dimension_numbers=(((1,), (1,)), ((), ())),
            preferred_element_type=jnp.float32)

        a = a_ref[0, 0]
        for i in range(nblk):
            s = i % 2
            pltpu.make_async_copy(adj_ref.at[pl.ds(i * blk, blk), :],
                                  abuf_ref.at[s], asem.at[s]).wait()
            acc = jnp.dot(abuf_ref[s], fts_ref[...],
                          preferred_element_type=jnp.float32)
            if i + 2 < nblk:
                pltpu.make_async_copy(
                    adj_ref.at[pl.ds((i + 2) * blk, blk), :],
                    abuf_ref.at[s], asem.at[s]).start()
            acc = acc + b_ref[...]
            if i >= 2:
                pltpu.make_async_copy(obuf_ref.at[s],
                                      out_ref.at[pl.ds((i - 2) * blk, blk), :],
                                      osem.at[s]).wait()
            obuf_ref[s] = jnp.where(acc >= 0, acc, a * acc)
            pltpu.make_async_copy(obuf_ref.at[s],
                                  out_ref.at[pl.ds(i * blk, blk), :],
                                  osem.at[s]).start()
        for i in range(max(nblk - 2, 0), nblk):
            s = i % 2
            pltpu.make_async_copy(obuf_ref.at[s],
                                  out_ref.at[pl.ds(i * blk, blk), :],
                                  osem.at[s]).wait()
    return _body


@jax.jit
def kernel(seq, adj, W, b, prelu_a):
    _, n, d_in = seq.shape
    d_out = W.shape[0]

    blk = 400 if n % 400 == 0 else n
    nblk = n // blk

    out = pl.pallas_call(
        _make_body(nblk, blk),
        in_specs=[
            pl.BlockSpec(memory_space=pltpu.VMEM),
            pl.BlockSpec(memory_space=pltpu.VMEM),
            pl.BlockSpec(memory_space=pltpu.VMEM),
            pl.BlockSpec(memory_space=pltpu.SMEM),
            pl.BlockSpec(memory_space=pl.ANY),
        ],
        out_specs=pl.BlockSpec(memory_space=pl.ANY),
        out_shape=jax.ShapeDtypeStruct((n, d_out), jnp.float32),
        scratch_shapes=[
            pltpu.VMEM((n, d_out), jnp.float32),
            pltpu.VMEM((2, blk, n), jnp.float32),
            pltpu.VMEM((2, blk, d_out), jnp.float32),
            pltpu.SemaphoreType.DMA((2,)),
            pltpu.SemaphoreType.DMA((2,)),
        ],
    )(seq.reshape(n, d_in), W, b.reshape(1, d_out), prelu_a.reshape(1, 1),
      adj.reshape(n, n))

    return out[None]


# speedup vs baseline: 1.0698x; 1.0698x over previous
import functools

import jax
import jax.numpy as jnp
from jax.experimental import pallas as pl
from jax.experimental.pallas import tpu as pltpu


def _gcn_kernel(seq_ref, w_ref, adj_ref, b_ref, a_ref, out_ref, fts_ref):
    @pl.when(pl.program_id(0) == 0)
    def _():
        fts_ref[...] = jax.lax.dot_general(
            seq_ref[...], w_ref[...],
            dimension_numbers=(((1,), (1,)), ((), ())),
            preferred_element_type=jnp.float32)

    acc = jnp.dot(adj_ref[...].astype(jnp.bfloat16),
                  fts_ref[...].astype(jnp.bfloat16),
                  preferred_element_type=jnp.float32)
    acc = acc + b_ref[...]
    out_ref[...] = jnp.where(acc >= 0, acc, a_ref[0, 0] * acc)


@jax.jit
def kernel(seq, adj, W, b, prelu_a):
    _, n, d_in = seq.shape
    d_out = W.shape[0]

    blk = 400 if n % 400 == 0 else n
    grid = n // blk

    out = pl.pallas_call(
        _gcn_kernel,
        grid=(grid,),
        in_specs=[
            pl.BlockSpec((n, d_in), lambda i: (0, 0)),
            pl.BlockSpec((d_out, d_in), lambda i: (0, 0)),
            pl.BlockSpec((blk, n), lambda i: (i, 0)),
            pl.BlockSpec((1, d_out), lambda i: (0, 0)),
            pl.BlockSpec(memory_space=pltpu.SMEM),
        ],
        out_specs=pl.BlockSpec((blk, d_out), lambda i: (i, 0)),
        out_shape=jax.ShapeDtypeStruct((n, d_out), jnp.float32),
        scratch_shapes=[pltpu.VMEM((n, d_out), jnp.float32)],
    )(seq.reshape(n, d_in), W, adj.reshape(n, n), b.reshape(1, d_out),
      prelu_a.reshape(1, 1))

    return out[None]
